# baseline (speedup 1.0000x reference)
import functools

import jax
import jax.numpy as jnp
from jax import lax
from jax.experimental import pallas as pl
from jax.experimental.pallas import tpu as pltpu
from jax.experimental.pallas import tpu_sc as plsc

N = 50000
E = 800000
VOCAB = 100000
EMB = 32
HID = 64
NCLS = 2
G = 128

NCORE = 2
NSUB = 16
NW = NCORE * NSUB
C = 128
NBUF = 4

NPT = 1664
NCH_N = NPT // C
N_PAD = NW * NPT
EPT1 = 25088
NCH_E1 = EPT1 // C
E_PAD = NW * EPT1
EPT2 = E_PAD // NSUB
NCH_E2 = EPT2 // C
GRP = 14
NG1 = NCH_E1 // GRP
NPH = 4
GRP2 = 7
NGP = NCH_E1 // NPH // GRP2
RPT = N_PAD // NSUB
ZROWS = 64
GP = G + 16
PW = 80
BLK = 2048
NBLK = N_PAD // BLK

_mesh = plsc.VectorSubcoreMesh(
    core_axis_name="c", subcore_axis_name="s", num_cores=NCORE,
    num_subcores=NSUB)

_SEG_SCRATCH = [
    pltpu.VMEM((GRP, C), jnp.int32),
    pltpu.VMEM((GRP, C), jnp.int32),
    pltpu.VMEM((NBUF, C, EMB), jnp.float32),
    pltpu.VMEM((ZROWS, EMB), jnp.float32),
    pltpu.VMEM_SHARED((N_PAD, EMB), jnp.float32),
] + [pltpu.SemaphoreType.DMA] * (2 * NBUF)


def _zero_vmem(buf):
    rows, width = buf.shape
    lanes = 32 if buf.dtype == jnp.bfloat16 else 16
    zv = jnp.zeros((lanes,), buf.dtype)

    def body(i, carry):
        for k in range(width // lanes):
            buf[i, pl.ds(k * lanes, lanes)] = zv
        return carry

    lax.fori_loop(0, rows, body, 0)


def _zero_shared(acc, zbuf, row0, nrows):
    zrows = zbuf.shape[0]

    def body(k, carry):
        pltpu.sync_copy(zbuf, acc.at[pl.ds(row0 + k * zrows, zrows)])
        return carry

    lax.fori_loop(0, nrows // zrows, body, 0)


def _edge_pipeline(hsrc, src_hbm, dst_hbm, srcbuf, dstbuf, rows, gsems,
                   ssems, acc, n_groups, grp=GRP, base=0):
    def group(g, carry):
        pltpu.sync_copy(src_hbm.at[pl.ds(base + g * grp, grp)], srcbuf)
        pltpu.sync_copy(dst_hbm.at[pl.ds(base + g * grp, grp)], dstbuf)
        gd = [None] * grp
        sd = [None] * grp
        for j in range(2):
            gd[j] = pltpu.async_copy(
                hsrc.at[srcbuf.at[j]], rows.at[j % NBUF], gsems[j % NBUF])
        for j in range(grp):
            jn = j + 2
            if jn < grp:
                if jn >= NBUF:
                    sd[jn - NBUF].wait()
                gd[jn] = pltpu.async_copy(
                    hsrc.at[srcbuf.at[jn]], rows.at[jn % NBUF],
                    gsems[jn % NBUF])
            gd[j].wait()
            sd[j] = pltpu.async_copy(
                rows.at[j % NBUF], acc.at[dstbuf.at[j]], ssems[j % NBUF],
                add=True)
        for j in range(grp - NBUF, grp):
            sd[j].wait()
        return carry

    lax.fori_loop(0, n_groups, group, 0)


@functools.partial(
    pl.kernel,
    out_type=jax.ShapeDtypeStruct((N_PAD, EMB), jnp.float32),
    mesh=_mesh,
    compiler_params=pltpu.CompilerParams(use_tc_tiling_on_sc=False),
    scratch_types=[
        pltpu.VMEM((NCH_N, C), jnp.int32),
        pltpu.VMEM((NPT, EMB), jnp.float32),
        pltpu.SemaphoreType.DMA,
    ],
)
def _emb_gather(table, x_r, out, idxbuf, rows, sem):
    c = lax.axis_index("c")
    s = lax.axis_index("s")
    wid = s * NCORE + c
    pltpu.sync_copy(x_r.at[wid], idxbuf)
    gd = []
    for j in range(NCH_N):
        gd.append(pltpu.async_copy(
            table.at[idxbuf.at[j]], rows.at[pl.ds(j * C, C)], sem))
    for d in gd:
        d.wait()
    pltpu.sync_copy(rows, out.at[pl.ds(wid * NPT, NPT)])


@functools.partial(
    pl.kernel,
    out_type=jax.ShapeDtypeStruct((NCORE, N_PAD, EMB), jnp.float32),
    mesh=_mesh,
    compiler_params=pltpu.CompilerParams(use_tc_tiling_on_sc=False),
    scratch_types=_SEG_SCRATCH,
)
def _seg1(h0, src_r, dst_r, out, srcbuf, dstbuf, rows, zbuf, acc, *sems):
    c = lax.axis_index("c")
    s = lax.axis_index("s")
    wid = s * NCORE + c
    _zero_vmem(zbuf)
    _zero_shared(acc, zbuf, s * RPT, RPT)
    plsc.subcore_barrier()
    _edge_pipeline(h0, src_r.at[wid], dst_r.at[wid], srcbuf, dstbuf, rows,
                   sems[:NBUF], sems[NBUF:], acc, NG1)
    plsc.subcore_barrier()
    pltpu.sync_copy(acc.at[pl.ds(s * RPT, RPT)],
                    out.at[c].at[pl.ds(s * RPT, RPT)])


@functools.partial(
    pl.kernel,
    out_type=jax.ShapeDtypeStruct((NPH * NCORE, N_PAD, HID), jnp.bfloat16),
    mesh=_mesh,
    compiler_params=pltpu.CompilerParams(use_tc_tiling_on_sc=False),
    scratch_types=[
        pltpu.VMEM((GRP2, C), jnp.int32),
        pltpu.VMEM((GRP2, C), jnp.int32),
        pltpu.VMEM((NBUF, C, HID), jnp.bfloat16),
        pltpu.VMEM((ZROWS, HID), jnp.bfloat16),
        pltpu.VMEM_SHARED((N_PAD, HID), jnp.bfloat16),
    ] + [pltpu.SemaphoreType.DMA] * (2 * NBUF),
)
def _seg2(h1b, src_r, dst_r, out, srcbuf, dstbuf, rows, zbuf, acc, *sems):
    c = lax.axis_index("c")
    s = lax.axis_index("s")
    wid = s * NCORE + c
    _zero_vmem(zbuf)
    _zero_shared(acc, zbuf, s * RPT, RPT)
    for p in range(NPH):
        plsc.subcore_barrier()
        _edge_pipeline(h1b, src_r.at[wid], dst_r.at[wid], srcbuf, dstbuf,
                       rows, sems[:NBUF], sems[NBUF:], acc, NGP,
                       grp=GRP2, base=p * NGP * GRP2)
        plsc.subcore_barrier()
        pltpu.sync_copy(acc.at[pl.ds(s * RPT, RPT)],
                        out.at[p * NCORE + c].at[pl.ds(s * RPT, RPT)])
        if p + 1 < NPH:
            _zero_shared(acc, zbuf, s * RPT, RPT)


def _mm1_body(aref, h0ref, wrel, wroot, bref, oref, orefb):
    a = aref[0] + aref[1]
    r = (jnp.dot(a, wrel[...], preferred_element_type=jnp.float32)
         + jnp.dot(h0ref[...], wroot[...], preferred_element_type=jnp.float32)
         + bref[...])
    h1 = jnp.maximum(r, 0.0)
    oref[...] = h1
    orefb[...] = h1.astype(jnp.bfloat16)


def _mm1(agg1, h0, W1_rel, W1_root, b1):
    return pl.pallas_call(
        _mm1_body,
        grid=(NBLK,),
        in_specs=[
            pl.BlockSpec((NCORE, BLK, EMB), lambda i: (0, i, 0)),
            pl.BlockSpec((BLK, EMB), lambda i: (i, 0)),
            pl.BlockSpec((EMB, HID), lambda i: (0, 0)),
            pl.BlockSpec((EMB, HID), lambda i: (0, 0)),
            pl.BlockSpec((1, HID), lambda i: (0, 0)),
        ],
        out_specs=[
            pl.BlockSpec((BLK, HID), lambda i: (i, 0)),
            pl.BlockSpec((BLK, HID), lambda i: (i, 0)),
        ],
        out_shape=[
            jax.ShapeDtypeStruct((N_PAD, HID), jnp.float32),
            jax.ShapeDtypeStruct((N_PAD, HID), jnp.bfloat16),
        ],
    )(agg1, h0, W1_rel, W1_root, b1)


def _mm2_pool_body(a2ref, h1ref, bat, wrel, wroot, bref, wl, bl, oref, acc):
    i = pl.program_id(0)

    @pl.when(i == 0)
    def _():
        acc[...] = jnp.zeros((G, HID + 1), jnp.float32)

    a = sum(a2ref[q].astype(jnp.float32) for q in range(1, NPH * NCORE))
    a = a + a2ref[0].astype(jnp.float32)
    r = (jnp.dot(a, wrel[...], preferred_element_type=jnp.float32)
         + jnp.dot(h1ref[...], wroot[...], preferred_element_type=jnp.float32)
         + bref[...])
    h2 = jnp.maximum(r, 0.0)
    sel = (bat[...] == lax.broadcasted_iota(jnp.int32, (BLK, G), 1))
    sel = sel.astype(jnp.float32)
    h2o = jnp.concatenate([h2, jnp.ones((BLK, 1), jnp.float32)], axis=1)
    acc[...] += lax.dot_general(sel, h2o, (((0,), (0,)), ((), ())),
                                preferred_element_type=jnp.float32)

    @pl.when(i == NBLK - 1)
    def _():
        su = acc[...]
        pooled = su[:, :HID] / jnp.maximum(su[:, HID:HID + 1], 1.0)
        oref[...] = (jnp.dot(pooled, wl[...],
                             preferred_element_type=jnp.float32) + bl[...])


def _mm2_pool(agg2, h1, batch_p, W2_rel, W2_root, b2, W_lin, b_lin):
    return pl.pallas_call(
        _mm2_pool_body,
        grid=(NBLK,),
        in_specs=[
            pl.BlockSpec((NPH * NCORE, BLK, HID), lambda i: (0, i, 0)),
            pl.BlockSpec((BLK, HID), lambda i: (i, 0)),
            pl.BlockSpec((BLK, 1), lambda i: (i, 0)),
            pl.BlockSpec((HID, HID), lambda i: (0, 0)),
            pl.BlockSpec((HID, HID), lambda i: (0, 0)),
            pl.BlockSpec((1, HID), lambda i: (0, 0)),
            pl.BlockSpec((HID, NCLS), lambda i: (0, 0)),
            pl.BlockSpec((1, NCLS), lambda i: (0, 0)),
        ],
        out_specs=pl.BlockSpec((G, NCLS), lambda i: (0, 0)),
        out_shape=jax.ShapeDtypeStruct((G, NCLS), jnp.float32),
        scratch_shapes=[pltpu.VMEM((G, HID + 1), jnp.float32)],
    )(agg2, h1, batch_p, W2_rel, W2_root, b2, W_lin, b_lin)


def kernel(x, edge_index, batch, emb_table, W1_rel, W1_root, b1,
           W2_rel, W2_root, b2, W_lin, b_lin):
    i32 = jnp.int32
    table = emb_table.at[0].set(0.0)
    x_r = jnp.concatenate(
        [x.astype(i32), jnp.zeros((N_PAD - N,), i32)]).reshape(NW, NCH_N, C)
    src = edge_index[0].astype(i32)
    dst = edge_index[1].astype(i32)
    src_p = jnp.concatenate([src, jnp.zeros((E_PAD - E,), i32)])
    dst_p = jnp.concatenate([dst, jnp.full((E_PAD - E,), N, i32)])
    src1_r = src_p.reshape(NW, NCH_E1, C)
    dst1_r = dst_p.reshape(NW, NCH_E1, C)
    batch_p = jnp.concatenate(
        [batch.astype(i32), jnp.full((N_PAD - N,), G, i32)]
    ).reshape(N_PAD, 1)

    h0 = _emb_gather(table, x_r)
    agg1 = _seg1(h0, src1_r, dst1_r)
    h1, h1b = _mm1(agg1, h0, W1_rel, W1_root, b1.reshape(1, HID))
    agg2 = _seg2(h1b, src1_r, dst1_r)
    return _mm2_pool(agg2, h1, batch_p, W2_rel, W2_root, b2.reshape(1, HID),
                     W_lin, b_lin.reshape(1, NCLS))


# speedup vs baseline: 1.1266x; 1.1266x over previous
import functools

import jax
import jax.numpy as jnp
from jax import lax
from jax.experimental import pallas as pl
from jax.experimental.pallas import tpu as pltpu
from jax.experimental.pallas import tpu_sc as plsc

N = 50000
E = 800000
VOCAB = 100000
EMB = 32
HID = 64
NCLS = 2
G = 128

NCORE = 2
NSUB = 16
NW = NCORE * NSUB
C = 128
NBUF = 4

NPT = 1664
NCH_N = NPT // C
N_PAD = NW * NPT
EPT1 = 25088
NCH_E1 = EPT1 // C
E_PAD = NW * EPT1
NPH = 2
GRP = 7
NGP = NCH_E1 // NPH // GRP
RPT = N_PAD // NSUB
ZROWS = 64
GP = G + 16
PW = 80
BLK = 2048
NBLK = N_PAD // BLK

_mesh = plsc.VectorSubcoreMesh(
    core_axis_name="c", subcore_axis_name="s", num_cores=NCORE,
    num_subcores=NSUB)

def _seg_scratch(width):
    return [
        pltpu.VMEM((GRP, C), jnp.int32),
        pltpu.VMEM((GRP, C), jnp.int32),
        pltpu.VMEM((NBUF, C, width), jnp.bfloat16),
        pltpu.VMEM((ZROWS, width), jnp.bfloat16),
        pltpu.VMEM_SHARED((N_PAD, width), jnp.bfloat16),
    ] + [pltpu.SemaphoreType.DMA] * (2 * NBUF)


def _zero_vmem(buf):
    rows, width = buf.shape
    lanes = 32 if buf.dtype == jnp.bfloat16 else 16
    zv = jnp.zeros((lanes,), buf.dtype)

    def body(i, carry):
        for k in range(width // lanes):
            buf[i, pl.ds(k * lanes, lanes)] = zv
        return carry

    lax.fori_loop(0, rows, body, 0)


def _zero_shared(acc, zbuf, row0, nrows):
    zrows = zbuf.shape[0]

    def body(k, carry):
        pltpu.sync_copy(zbuf, acc.at[pl.ds(row0 + k * zrows, zrows)])
        return carry

    lax.fori_loop(0, nrows // zrows, body, 0)


def _edge_pipeline(hsrc, src_hbm, dst_hbm, srcbuf, dstbuf, rows, gsems,
                   ssems, acc, n_groups, grp=GRP, base=0):
    def group(g, carry):
        pltpu.sync_copy(src_hbm.at[pl.ds(base + g * grp, grp)], srcbuf)
        pltpu.sync_copy(dst_hbm.at[pl.ds(base + g * grp, grp)], dstbuf)
        gd = [None] * grp
        sd = [None] * grp
        for j in range(2):
            gd[j] = pltpu.async_copy(
                hsrc.at[srcbuf.at[j]], rows.at[j % NBUF], gsems[j % NBUF])
        for j in range(grp):
            jn = j + 2
            if jn < grp:
                if jn >= NBUF:
                    sd[jn - NBUF].wait()
                gd[jn] = pltpu.async_copy(
                    hsrc.at[srcbuf.at[jn]], rows.at[jn % NBUF],
                    gsems[jn % NBUF])
            gd[j].wait()
            sd[j] = pltpu.async_copy(
                rows.at[j % NBUF], acc.at[dstbuf.at[j]], ssems[j % NBUF],
                add=True)
        for j in range(grp - NBUF, grp):
            sd[j].wait()
        return carry

    lax.fori_loop(0, n_groups, group, 0)


@functools.partial(
    pl.kernel,
    out_type=jax.ShapeDtypeStruct((N_PAD, EMB), jnp.bfloat16),
    mesh=_mesh,
    compiler_params=pltpu.CompilerParams(use_tc_tiling_on_sc=False),
    scratch_types=[
        pltpu.VMEM((NCH_N, C), jnp.int32),
        pltpu.VMEM((NPT, EMB), jnp.bfloat16),
        pltpu.SemaphoreType.DMA,
    ],
)
def _emb_gather(table, x_r, out, idxbuf, rows, sem):
    c = lax.axis_index("c")
    s = lax.axis_index("s")
    wid = s * NCORE + c
    pltpu.sync_copy(x_r.at[wid], idxbuf)
    gd = []
    for j in range(NCH_N):
        gd.append(pltpu.async_copy(
            table.at[idxbuf.at[j]], rows.at[pl.ds(j * C, C)], sem))
    for d in gd:
        d.wait()
    pltpu.sync_copy(rows, out.at[pl.ds(wid * NPT, NPT)])


@functools.partial(
    pl.kernel,
    out_type=jax.ShapeDtypeStruct((NPH * NCORE, N_PAD, EMB), jnp.bfloat16),
    mesh=_mesh,
    compiler_params=pltpu.CompilerParams(use_tc_tiling_on_sc=False),
    scratch_types=_seg_scratch(EMB),
)
def _seg1(h0b, src_r, dst_r, out, srcbuf, dstbuf, rows, zbuf, acc, *sems):
    c = lax.axis_index("c")
    s = lax.axis_index("s")
    wid = s * NCORE + c
    _zero_vmem(zbuf)
    _zero_shared(acc, zbuf, s * RPT, RPT)
    for p in range(NPH):
        plsc.subcore_barrier()
        _edge_pipeline(h0b, src_r.at[wid], dst_r.at[wid], srcbuf, dstbuf,
                       rows, sems[:NBUF], sems[NBUF:], acc, NGP,
                       base=p * NGP * GRP)
        plsc.subcore_barrier()
        pltpu.sync_copy(acc.at[pl.ds(s * RPT, RPT)],
                        out.at[p * NCORE + c].at[pl.ds(s * RPT, RPT)])
        if p + 1 < NPH:
            _zero_shared(acc, zbuf, s * RPT, RPT)


@functools.partial(
    pl.kernel,
    out_type=jax.ShapeDtypeStruct((NPH * NCORE, N_PAD, HID), jnp.bfloat16),
    mesh=_mesh,
    compiler_params=pltpu.CompilerParams(use_tc_tiling_on_sc=False),
    scratch_types=_seg_scratch(HID),
)
def _seg2(h1b, src_r, dst_r, out, srcbuf, dstbuf, rows, zbuf, acc, *sems):
    c = lax.axis_index("c")
    s = lax.axis_index("s")
    wid = s * NCORE + c
    _zero_vmem(zbuf)
    _zero_shared(acc, zbuf, s * RPT, RPT)
    for p in range(NPH):
        plsc.subcore_barrier()
        _edge_pipeline(h1b, src_r.at[wid], dst_r.at[wid], srcbuf, dstbuf,
                       rows, sems[:NBUF], sems[NBUF:], acc, NGP,
                       base=p * NGP * GRP)
        plsc.subcore_barrier()
        pltpu.sync_copy(acc.at[pl.ds(s * RPT, RPT)],
                        out.at[p * NCORE + c].at[pl.ds(s * RPT, RPT)])
        if p + 1 < NPH:
            _zero_shared(acc, zbuf, s * RPT, RPT)


def _mm1_body(aref, h0ref, wrel, wroot, bref, oref, orefb):
    a = sum(aref[q].astype(jnp.float32) for q in range(1, NPH * NCORE))
    a = a + aref[0].astype(jnp.float32)
    h0 = h0ref[...].astype(jnp.float32)
    r = (jnp.dot(a, wrel[...], preferred_element_type=jnp.float32)
         + jnp.dot(h0, wroot[...], preferred_element_type=jnp.float32)
         + bref[...])
    h1 = jnp.maximum(r, 0.0)
    oref[...] = h1
    orefb[...] = h1.astype(jnp.bfloat16)


def _mm1(agg1, h0b, W1_rel, W1_root, b1):
    return pl.pallas_call(
        _mm1_body,
        grid=(NBLK,),
        in_specs=[
            pl.BlockSpec((NPH * NCORE, BLK, EMB), lambda i: (0, i, 0)),
            pl.BlockSpec((BLK, EMB), lambda i: (i, 0)),
            pl.BlockSpec((EMB, HID), lambda i: (0, 0)),
            pl.BlockSpec((EMB, HID), lambda i: (0, 0)),
            pl.BlockSpec((1, HID), lambda i: (0, 0)),
        ],
        out_specs=[
            pl.BlockSpec((BLK, HID), lambda i: (i, 0)),
            pl.BlockSpec((BLK, HID), lambda i: (i, 0)),
        ],
        out_shape=[
            jax.ShapeDtypeStruct((N_PAD, HID), jnp.float32),
            jax.ShapeDtypeStruct((N_PAD, HID), jnp.bfloat16),
        ],
    )(agg1, h0b, W1_rel, W1_root, b1)


def _mm2_pool_body(a2ref, h1ref, bat, wrel, wroot, bref, wl, bl, oref, acc):
    i = pl.program_id(0)

    @pl.when(i == 0)
    def _():
        acc[...] = jnp.zeros((G, HID + 1), jnp.float32)

    a = sum(a2ref[q].astype(jnp.float32) for q in range(1, NPH * NCORE))
    a = a + a2ref[0].astype(jnp.float32)
    r = (jnp.dot(a, wrel[...], preferred_element_type=jnp.float32)
         + jnp.dot(h1ref[...], wroot[...], preferred_element_type=jnp.float32)
         + bref[...])
    h2 = jnp.maximum(r, 0.0)
    sel = (bat[...] == lax.broadcasted_iota(jnp.int32, (BLK, G), 1))
    sel = sel.astype(jnp.float32)
    h2o = jnp.concatenate([h2, jnp.ones((BLK, 1), jnp.float32)], axis=1)
    acc[...] += lax.dot_general(sel, h2o, (((0,), (0,)), ((), ())),
                                preferred_element_type=jnp.float32)

    @pl.when(i == NBLK - 1)
    def _():
        su = acc[...]
        pooled = su[:, :HID] / jnp.maximum(su[:, HID:HID + 1], 1.0)
        oref[...] = (jnp.dot(pooled, wl[...],
                             preferred_element_type=jnp.float32) + bl[...])


def _mm2_pool(agg2, h1, batch_p, W2_rel, W2_root, b2, W_lin, b_lin):
    return pl.pallas_call(
        _mm2_pool_body,
        grid=(NBLK,),
        in_specs=[
            pl.BlockSpec((NPH * NCORE, BLK, HID), lambda i: (0, i, 0)),
            pl.BlockSpec((BLK, HID), lambda i: (i, 0)),
            pl.BlockSpec((BLK, 1), lambda i: (i, 0)),
            pl.BlockSpec((HID, HID), lambda i: (0, 0)),
            pl.BlockSpec((HID, HID), lambda i: (0, 0)),
            pl.BlockSpec((1, HID), lambda i: (0, 0)),
            pl.BlockSpec((HID, NCLS), lambda i: (0, 0)),
            pl.BlockSpec((1, NCLS), lambda i: (0, 0)),
        ],
        out_specs=pl.BlockSpec((G, NCLS), lambda i: (0, 0)),
        out_shape=jax.ShapeDtypeStruct((G, NCLS), jnp.float32),
        scratch_shapes=[pltpu.VMEM((G, HID + 1), jnp.float32)],
    )(agg2, h1, batch_p, W2_rel, W2_root, b2, W_lin, b_lin)


def kernel(x, edge_index, batch, emb_table, W1_rel, W1_root, b1,
           W2_rel, W2_root, b2, W_lin, b_lin):
    i32 = jnp.int32
    table_b = emb_table.at[0].set(0.0).astype(jnp.bfloat16)
    x_r = jnp.concatenate(
        [x.astype(i32), jnp.zeros((N_PAD - N,), i32)]).reshape(NW, NCH_N, C)
    src = edge_index[0].astype(i32)
    dst = edge_index[1].astype(i32)
    src_p = jnp.concatenate([src, jnp.zeros((E_PAD - E,), i32)])
    dst_p = jnp.concatenate([dst, jnp.full((E_PAD - E,), N, i32)])
    src1_r = src_p.reshape(NW, NCH_E1, C)
    dst1_r = dst_p.reshape(NW, NCH_E1, C)
    batch_p = jnp.concatenate(
        [batch.astype(i32), jnp.full((N_PAD - N,), G, i32)]
    ).reshape(N_PAD, 1)

    h0b = _emb_gather(table_b, x_r)
    agg1 = _seg1(h0b, src1_r, dst1_r)
    h1, h1b = _mm1(agg1, h0b, W1_rel, W1_root, b1.reshape(1, HID))
    agg2 = _seg2(h1b, src1_r, dst1_r)
    return _mm2_pool(agg2, h1, batch_p, W2_rel, W2_root, b2.reshape(1, HID),
                     W_lin, b_lin.reshape(1, NCLS))


# speedup vs baseline: 1.3855x; 1.2298x over previous
import functools

import jax
import jax.numpy as jnp
from jax import lax
from jax.experimental import pallas as pl
from jax.experimental.pallas import tpu as pltpu
from jax.experimental.pallas import tpu_sc as plsc

N = 50000
E = 800000
VOCAB = 100000
EMB = 32
HID = 64
NCLS = 2
G = 128

NCORE = 2
NSUB = 16
NW = NCORE * NSUB
C = 128
NBUF = 4

NPT = 1664
NCH_N = NPT // C
N_PAD = NW * NPT
EPT1 = 25088
NCH_E1 = EPT1 // C
E_PAD = NW * EPT1
NPH = 1
GRP = 7
NGP = NCH_E1 // NPH // GRP
RPT = N_PAD // NSUB
ZROWS = 64
GP = G + 16
PW = 80
BLK = 2048
NBLK = N_PAD // BLK

_mesh = plsc.VectorSubcoreMesh(
    core_axis_name="c", subcore_axis_name="s", num_cores=NCORE,
    num_subcores=NSUB)

def _seg_scratch(width):
    return [
        pltpu.VMEM((GRP, C), jnp.int32),
        pltpu.VMEM((GRP, C), jnp.int32),
        pltpu.VMEM((NBUF, C, width), jnp.bfloat16),
        pltpu.VMEM((ZROWS, width), jnp.bfloat16),
        pltpu.VMEM_SHARED((N_PAD, width), jnp.bfloat16),
    ] + [pltpu.SemaphoreType.DMA] * (2 * NBUF)


def _zero_vmem(buf):
    rows, width = buf.shape
    lanes = 32 if buf.dtype == jnp.bfloat16 else 16
    zv = jnp.zeros((lanes,), buf.dtype)

    def body(i, carry):
        for k in range(width // lanes):
            buf[i, pl.ds(k * lanes, lanes)] = zv
        return carry

    lax.fori_loop(0, rows, body, 0)


def _zero_shared(acc, zbuf, row0, nrows):
    zrows = zbuf.shape[0]

    def body(k, carry):
        pltpu.sync_copy(zbuf, acc.at[pl.ds(row0 + k * zrows, zrows)])
        return carry

    lax.fori_loop(0, nrows // zrows, body, 0)


def _edge_pipeline(hsrc, src_hbm, dst_hbm, srcbuf, dstbuf, rows, gsems,
                   ssems, acc, n_groups, grp=GRP, base=0):
    def group(g, carry):
        pltpu.sync_copy(src_hbm.at[pl.ds(base + g * grp, grp)], srcbuf)
        pltpu.sync_copy(dst_hbm.at[pl.ds(base + g * grp, grp)], dstbuf)
        gd = [None] * grp
        sd = [None] * grp
        for j in range(2):
            gd[j] = pltpu.async_copy(
                hsrc.at[srcbuf.at[j]], rows.at[j % NBUF], gsems[j % NBUF])
        for j in range(grp):
            jn = j + 2
            if jn < grp:
                if jn >= NBUF:
                    sd[jn - NBUF].wait()
                gd[jn] = pltpu.async_copy(
                    hsrc.at[srcbuf.at[jn]], rows.at[jn % NBUF],
                    gsems[jn % NBUF])
            gd[j].wait()
            sd[j] = pltpu.async_copy(
                rows.at[j % NBUF], acc.at[dstbuf.at[j]], ssems[j % NBUF],
                add=True)
        for j in range(grp - NBUF, grp):
            sd[j].wait()
        return carry

    lax.fori_loop(0, n_groups, group, 0)


@functools.partial(
    pl.kernel,
    out_type=jax.ShapeDtypeStruct((N_PAD, EMB), jnp.bfloat16),
    mesh=_mesh,
    compiler_params=pltpu.CompilerParams(use_tc_tiling_on_sc=False),
    scratch_types=[
        pltpu.VMEM((NCH_N, C), jnp.int32),
        pltpu.VMEM((NPT, EMB), jnp.bfloat16),
        pltpu.SemaphoreType.DMA,
    ],
)
def _emb_gather(table, x_r, out, idxbuf, rows, sem):
    c = lax.axis_index("c")
    s = lax.axis_index("s")
    wid = s * NCORE + c
    pltpu.sync_copy(x_r.at[wid], idxbuf)
    gd = []
    for j in range(NCH_N):
        gd.append(pltpu.async_copy(
            table.at[idxbuf.at[j]], rows.at[pl.ds(j * C, C)], sem))
    for d in gd:
        d.wait()
    pltpu.sync_copy(rows, out.at[pl.ds(wid * NPT, NPT)])


@functools.partial(
    pl.kernel,
    out_type=jax.ShapeDtypeStruct((NPH * NCORE, N_PAD, EMB), jnp.bfloat16),
    mesh=_mesh,
    compiler_params=pltpu.CompilerParams(use_tc_tiling_on_sc=False),
    scratch_types=_seg_scratch(EMB),
)
def _seg1(h0b, src_r, dst_r, out, srcbuf, dstbuf, rows, zbuf, acc, *sems):
    c = lax.axis_index("c")
    s = lax.axis_index("s")
    wid = s * NCORE + c
    _zero_vmem(zbuf)
    _zero_shared(acc, zbuf, s * RPT, RPT)
    for p in range(NPH):
        plsc.subcore_barrier()
        _edge_pipeline(h0b, src_r.at[wid], dst_r.at[wid], srcbuf, dstbuf,
                       rows, sems[:NBUF], sems[NBUF:], acc, NGP,
                       base=p * NGP * GRP)
        plsc.subcore_barrier()
        pltpu.sync_copy(acc.at[pl.ds(s * RPT, RPT)],
                        out.at[p * NCORE + c].at[pl.ds(s * RPT, RPT)])
        if p + 1 < NPH:
            _zero_shared(acc, zbuf, s * RPT, RPT)


@functools.partial(
    pl.kernel,
    out_type=jax.ShapeDtypeStruct((NPH * NCORE, N_PAD, HID), jnp.bfloat16),
    mesh=_mesh,
    compiler_params=pltpu.CompilerParams(use_tc_tiling_on_sc=False),
    scratch_types=_seg_scratch(HID),
)
def _seg2(h1b, src_r, dst_r, out, srcbuf, dstbuf, rows, zbuf, acc, *sems):
    c = lax.axis_index("c")
    s = lax.axis_index("s")
    wid = s * NCORE + c
    _zero_vmem(zbuf)
    _zero_shared(acc, zbuf, s * RPT, RPT)
    for p in range(NPH):
        plsc.subcore_barrier()
        _edge_pipeline(h1b, src_r.at[wid], dst_r.at[wid], srcbuf, dstbuf,
                       rows, sems[:NBUF], sems[NBUF:], acc, NGP,
                       base=p * NGP * GRP)
        plsc.subcore_barrier()
        pltpu.sync_copy(acc.at[pl.ds(s * RPT, RPT)],
                        out.at[p * NCORE + c].at[pl.ds(s * RPT, RPT)])
        if p + 1 < NPH:
            _zero_shared(acc, zbuf, s * RPT, RPT)


def _mm1_body(aref, h0ref, wrel, wroot, bref, oref, orefb):
    a = sum(aref[q].astype(jnp.float32) for q in range(1, NPH * NCORE))
    a = a + aref[0].astype(jnp.float32)
    h0 = h0ref[...].astype(jnp.float32)
    r = (jnp.dot(a, wrel[...], preferred_element_type=jnp.float32)
         + jnp.dot(h0, wroot[...], preferred_element_type=jnp.float32)
         + bref[...])
    h1 = jnp.maximum(r, 0.0)
    oref[...] = h1
    orefb[...] = h1.astype(jnp.bfloat16)


def _mm1(agg1, h0b, W1_rel, W1_root, b1):
    return pl.pallas_call(
        _mm1_body,
        grid=(NBLK,),
        in_specs=[
            pl.BlockSpec((NPH * NCORE, BLK, EMB), lambda i: (0, i, 0)),
            pl.BlockSpec((BLK, EMB), lambda i: (i, 0)),
            pl.BlockSpec((EMB, HID), lambda i: (0, 0)),
            pl.BlockSpec((EMB, HID), lambda i: (0, 0)),
            pl.BlockSpec((1, HID), lambda i: (0, 0)),
        ],
        out_specs=[
            pl.BlockSpec((BLK, HID), lambda i: (i, 0)),
            pl.BlockSpec((BLK, HID), lambda i: (i, 0)),
        ],
        out_shape=[
            jax.ShapeDtypeStruct((N_PAD, HID), jnp.float32),
            jax.ShapeDtypeStruct((N_PAD, HID), jnp.bfloat16),
        ],
    )(agg1, h0b, W1_rel, W1_root, b1)


def _mm2_pool_body(a2ref, h1ref, bat, wrel, wroot, bref, wl, bl, oref, acc):
    i = pl.program_id(0)

    @pl.when(i == 0)
    def _():
        acc[...] = jnp.zeros((G, HID + 1), jnp.float32)

    a = sum(a2ref[q].astype(jnp.float32) for q in range(1, NPH * NCORE))
    a = a + a2ref[0].astype(jnp.float32)
    r = (jnp.dot(a, wrel[...], preferred_element_type=jnp.float32)
         + jnp.dot(h1ref[...], wroot[...], preferred_element_type=jnp.float32)
         + bref[...])
    h2 = jnp.maximum(r, 0.0)
    sel = (bat[...] == lax.broadcasted_iota(jnp.int32, (BLK, G), 1))
    sel = sel.astype(jnp.float32)
    h2o = jnp.concatenate([h2, jnp.ones((BLK, 1), jnp.float32)], axis=1)
    acc[...] += lax.dot_general(sel, h2o, (((0,), (0,)), ((), ())),
                                preferred_element_type=jnp.float32)

    @pl.when(i == NBLK - 1)
    def _():
        su = acc[...]
        pooled = su[:, :HID] / jnp.maximum(su[:, HID:HID + 1], 1.0)
        oref[...] = (jnp.dot(pooled, wl[...],
                             preferred_element_type=jnp.float32) + bl[...])


def _mm2_pool(agg2, h1, batch_p, W2_rel, W2_root, b2, W_lin, b_lin):
    return pl.pallas_call(
        _mm2_pool_body,
        grid=(NBLK,),
        in_specs=[
            pl.BlockSpec((NPH * NCORE, BLK, HID), lambda i: (0, i, 0)),
            pl.BlockSpec((BLK, HID), lambda i: (i, 0)),
            pl.BlockSpec((BLK, 1), lambda i: (i, 0)),
            pl.BlockSpec((HID, HID), lambda i: (0, 0)),
            pl.BlockSpec((HID, HID), lambda i: (0, 0)),
            pl.BlockSpec((1, HID), lambda i: (0, 0)),
            pl.BlockSpec((HID, NCLS), lambda i: (0, 0)),
            pl.BlockSpec((1, NCLS), lambda i: (0, 0)),
        ],
        out_specs=pl.BlockSpec((G, NCLS), lambda i: (0, 0)),
        out_shape=jax.ShapeDtypeStruct((G, NCLS), jnp.float32),
        scratch_shapes=[pltpu.VMEM((G, HID + 1), jnp.float32)],
    )(agg2, h1, batch_p, W2_rel, W2_root, b2, W_lin, b_lin)


def kernel(x, edge_index, batch, emb_table, W1_rel, W1_root, b1,
           W2_rel, W2_root, b2, W_lin, b_lin):
    i32 = jnp.int32
    table_b = emb_table.at[0].set(0.0).astype(jnp.bfloat16)
    x_r = jnp.concatenate(
        [x.astype(i32), jnp.zeros((N_PAD - N,), i32)]).reshape(NW, NCH_N, C)
    src = edge_index[0].astype(i32)
    dst = edge_index[1].astype(i32)
    src_p = jnp.concatenate([src, jnp.zeros((E_PAD - E,), i32)])
    dst_p = jnp.concatenate([dst, jnp.full((E_PAD - E,), N, i32)])
    src1_r = src_p.reshape(NW, NCH_E1, C)
    dst1_r = dst_p.reshape(NW, NCH_E1, C)
    batch_p = jnp.concatenate(
        [batch.astype(i32), jnp.full((N_PAD - N,), G, i32)]
    ).reshape(N_PAD, 1)

    h0b = _emb_gather(table_b, x_r)
    agg1 = _seg1(h0b, src1_r, dst1_r)
    h1, h1b = _mm1(agg1, h0b, W1_rel, W1_root, b1.reshape(1, HID))
    agg2 = _seg2(h1b, src1_r, dst1_r)
    return _mm2_pool(agg2, h1, batch_p, W2_rel, W2_root, b2.reshape(1, HID),
                     W_lin, b_lin.reshape(1, NCLS))
